# baseline (speedup 1.0000x reference)
import functools

import jax
import jax.numpy as jnp
from jax import lax
from jax.experimental import pallas as pl
from jax.experimental.pallas import tpu as pltpu
from jax.experimental.pallas import tpu_sc as plsc

_DIM = 32
_NC = 2
_NS = 16
_NW = _NC * _NS
_K = 128
_NBUF = 8
_DELTA = 4


@functools.lru_cache(maxsize=None)
def _build(n, m, vocab):
    batch = n * m
    assert batch % (_NW * _K) == 0
    bpw = batch // _NW
    nch = bpw // _K
    cpc = n // _K
    assert (nch - _NBUF) % _NBUF == 0
    mesh = plsc.VectorSubcoreMesh(core_axis_name="c", subcore_axis_name="s")

    @functools.partial(
        pl.kernel,
        mesh=mesh,
        out_type=jax.ShapeDtypeStruct((m, n // 4, 4, _DIM), jnp.float32),
        scratch_types=(
            [pltpu.VMEM((nch, _K), jnp.int32),
             pltpu.VMEM((_NBUF, _K, _DIM), jnp.float32)]
            + [pltpu.SemaphoreType.DMA] * (2 * _NBUF)
        ),
        compiler_params=pltpu.CompilerParams(use_tc_tiling_on_sc=False),
    )
    def emb(idx_hbm, table_hbm, out_hbm, idx_v, rows_v, *sems):
        gsem = sems[:_NBUF]
        osem = sems[_NBUF:]
        wid = lax.axis_index("s") * _NC + lax.axis_index("c")
        pltpu.sync_copy(idx_hbm.at[wid], idx_v)

        def remap(it, carry):
            c = it // (_K // 16)
            k = it % (_K // 16)
            v = idx_v[c, pl.ds(k * 16, 16)]
            j = ((v & -8192)
                 + ((v & 2047) << 2)
                 + ((v >> 11) & 3))
            idx_v[c, pl.ds(k * 16, 16)] = j
            return carry

        lax.fori_loop(0, nch * (_K // 16), remap, 0)

        def start_gather(c, b):
            pltpu.async_copy(table_hbm.at[idx_v.at[c]], rows_v.at[b], gsem[b])

        def wait_gather(c, b):
            pltpu.make_async_copy(
                table_hbm.at[idx_v.at[c]], rows_v.at[b], gsem[b]).wait()

        def _dst(c):
            gc = wid * nch + c
            col = gc // cpc
            cl = gc % cpc
            return out_hbm.at[col, pl.ds((cl % 8) * _K, _K), cl // 8]

        def start_out(c, b):
            pltpu.async_copy(rows_v.at[b], _dst(c), osem[b])

        def wait_out(c, b):
            pltpu.make_async_copy(rows_v.at[b], _dst(c), osem[b]).wait()

        for g in range(_DELTA):
            start_gather(g, g % _NBUF)
        for g in range(_DELTA, _NBUF):
            start_gather(g, g % _NBUF)
            c = g - _DELTA
            wait_gather(c, c % _NBUF)
            start_out(c, c % _NBUF)

        def group(i, carry):
            g0 = _NBUF + i * _NBUF
            for b in range(_NBUF):
                g = g0 + b
                wait_out(g - _NBUF, b)
                start_gather(g, b)
                cb = (b + _NBUF - _DELTA) % _NBUF
                wait_gather(g - _DELTA, cb)
                start_out(g - _DELTA, cb)
            return carry

        lax.fori_loop(0, (nch - _NBUF) // _NBUF, group, 0)

        for c in range(nch - _DELTA, nch):
            wait_gather(c, c % _NBUF)
            start_out(c, c % _NBUF)
        for c in range(nch - _NBUF, nch):
            wait_out(c, c % _NBUF)

    return emb


_TB = 8192


def _table_relayout(table_bc):
    vocab = table_bc.shape[1]
    grid = -(-vocab // _TB)

    def body(in_ref, out_ref):
        t = in_ref[...].T
        tp = jnp.pad(t, ((0, 0), (0, 128 - _DIM)))
        s = _TB // 4
        acc = tp[0:s]
        for q in range(1, 4):
            acc = acc + pltpu.roll(tp[q * s:(q + 1) * s], _DIM * q, axis=1)
        out_ref[...] = acc

    return pl.pallas_call(
        body,
        grid=(grid,),
        in_specs=[pl.BlockSpec((_DIM, _TB), lambda j: (0, j))],
        out_specs=pl.BlockSpec((_TB // 4, 128), lambda j: (j, 0)),
        out_shape=jax.ShapeDtypeStruct((grid * _TB // 4, 128),
                                       jnp.float32),
    )(table_bc)


_OB = 4


def _out_relayout(o):
    m, r128, _ = o.shape
    n = r128 * 128 // _DIM

    def body(in_ref, out_ref):
        for k in range(_OB):
            x = in_ref[k]
            xT = x.T
            x3 = xT.reshape(4, _DIM, r128)
            out_ref[k] = jnp.transpose(x3, (1, 0, 2)).reshape(_DIM, n)

    return pl.pallas_call(
        body,
        grid=(m // _OB,),
        in_specs=[pl.BlockSpec((_OB, r128, 128), lambda j: (j, 0, 0))],
        out_specs=pl.BlockSpec((_OB, _DIM, n), lambda j: (j, 0, 0)),
        out_shape=jax.ShapeDtypeStruct((m, _DIM, n), jnp.float32),
    )(o)


def kernel(x, table):
    n, m = x.shape
    batch = n * m
    idx = jnp.swapaxes(x, 0, 1).reshape(
        _NW, batch // _NW // _K, _K).astype(jnp.int32)
    t128 = _table_relayout(jnp.swapaxes(table, 0, 1))
    table_rm = t128.reshape(t128.shape[0] * 128 // _DIM, _DIM)
    out = _build(n, m, table_rm.shape[0])(idx, table_rm)
    ot = _out_relayout(out.reshape(m, n * _DIM // 128, 128))
    return jnp.transpose(ot, (2, 0, 1))


# speedup vs baseline: 1.3470x; 1.3470x over previous
import functools

import jax
import jax.numpy as jnp
from jax import lax
from jax.experimental import pallas as pl
from jax.experimental.pallas import tpu as pltpu
from jax.experimental.pallas import tpu_sc as plsc

_DIM = 32
_NC = 2
_NS = 16
_NW = _NC * _NS
_K = 128
_NBUF = 8
_DELTA = 4


@functools.lru_cache(maxsize=None)
def _build(n, m, vocab):
    batch = n * m
    assert batch % (_NW * _K) == 0
    bpw = batch // _NW
    nch = bpw // _K
    cpc = n // _K
    assert (nch - _NBUF) % _NBUF == 0
    mesh = plsc.VectorSubcoreMesh(core_axis_name="c", subcore_axis_name="s")

    @functools.partial(
        pl.kernel,
        mesh=mesh,
        out_type=jax.ShapeDtypeStruct((_NW, nch, _K, _DIM), jnp.float32),
        scratch_types=(
            [pltpu.VMEM((nch, _K), jnp.int32),
             pltpu.VMEM((_NBUF, _K, _DIM), jnp.float32)]
            + [pltpu.SemaphoreType.DMA] * (2 * _NBUF)
        ),
        compiler_params=pltpu.CompilerParams(use_tc_tiling_on_sc=False),
    )
    def emb(idx_hbm, table_hbm, out_hbm, idx_v, rows_v, *sems):
        gsem = sems[:_NBUF]
        osem = sems[_NBUF:]
        wid = lax.axis_index("s") * _NC + lax.axis_index("c")
        pltpu.sync_copy(idx_hbm.at[wid], idx_v)

        def remap(it, carry):
            c = it // (_K // 16)
            k = it % (_K // 16)
            v = idx_v[c, pl.ds(k * 16, 16)]
            j = ((v & -8192)
                 + ((v & 2047) << 2)
                 + ((v >> 11) & 3))
            idx_v[c, pl.ds(k * 16, 16)] = j
            return carry

        lax.fori_loop(0, nch * (_K // 16), remap, 0)

        def start_gather(c, b):
            pltpu.async_copy(table_hbm.at[idx_v.at[c]], rows_v.at[b], gsem[b])

        def wait_gather(c, b):
            pltpu.make_async_copy(
                table_hbm.at[idx_v.at[c]], rows_v.at[b], gsem[b]).wait()

        def start_out(c, b):
            pltpu.async_copy(rows_v.at[b], out_hbm.at[wid, c], osem[b])

        def wait_out(c, b):
            pltpu.make_async_copy(
                rows_v.at[b], out_hbm.at[wid, c], osem[b]).wait()

        for g in range(_DELTA):
            start_gather(g, g % _NBUF)
        for g in range(_DELTA, _NBUF):
            start_gather(g, g % _NBUF)
            c = g - _DELTA
            wait_gather(c, c % _NBUF)
            start_out(c, c % _NBUF)

        def group(i, carry):
            g0 = _NBUF + i * _NBUF
            for b in range(_NBUF):
                g = g0 + b
                wait_out(g - _NBUF, b)
                start_gather(g, b)
                cb = (b + _NBUF - _DELTA) % _NBUF
                wait_gather(g - _DELTA, cb)
                start_out(g - _DELTA, cb)
            return carry

        lax.fori_loop(0, (nch - _NBUF) // _NBUF, group, 0)

        for c in range(nch - _DELTA, nch):
            wait_gather(c, c % _NBUF)
            start_out(c, c % _NBUF)
        for c in range(nch - _NBUF, nch):
            wait_out(c, c % _NBUF)

    return emb


_TB = 8192


def _table_relayout(table_bc):
    vocab = table_bc.shape[1]
    grid = -(-vocab // _TB)

    def body(in_ref, out_ref):
        t = in_ref[...].T
        tp = jnp.pad(t, ((0, 0), (0, 128 - _DIM)))
        s = _TB // 4
        acc = tp[0:s]
        for q in range(1, 4):
            acc = acc + pltpu.roll(tp[q * s:(q + 1) * s], _DIM * q, axis=1)
        out_ref[...] = acc

    return pl.pallas_call(
        body,
        grid=(grid,),
        in_specs=[pl.BlockSpec((_DIM, _TB), lambda j: (0, j))],
        out_specs=pl.BlockSpec((_TB // 4, 128), lambda j: (j, 0)),
        out_shape=jax.ShapeDtypeStruct((grid * _TB // 4, 128),
                                       jnp.float32),
    )(table_bc)


_OB = 4


def _out_relayout(o):
    m, r128, _ = o.shape
    n = r128 * 128 // _DIM

    def body(in_ref, out_ref):
        for k in range(_OB):
            x = in_ref[k]
            pieces = []
            for q in range(4):
                r = pltpu.roll(x, 128 - _DIM * q, axis=1) if q else x
                pieces.append(r[:, None, :_DIM])
            emb = jnp.concatenate(pieces, axis=1).reshape(n, _DIM)
            out_ref[k] = emb.T

    return pl.pallas_call(
        body,
        grid=(m // _OB,),
        in_specs=[pl.BlockSpec((_OB, r128, 128), lambda j: (j, 0, 0))],
        out_specs=pl.BlockSpec((_OB, _DIM, n), lambda j: (j, 0, 0)),
        out_shape=jax.ShapeDtypeStruct((m, _DIM, n), jnp.float32),
    )(o)


def kernel(x, table):
    n, m = x.shape
    batch = n * m
    idx = jnp.swapaxes(x, 0, 1).reshape(
        _NW, batch // _NW // _K, _K).astype(jnp.int32)
    t128 = _table_relayout(jnp.swapaxes(table, 0, 1))
    table_rm = t128.reshape(t128.shape[0] * 128 // _DIM, _DIM)
    out = _build(n, m, table_rm.shape[0])(idx, table_rm)
    ot = _out_relayout(out.reshape(m, n * _DIM // 128, 128))
    return jnp.transpose(ot, (2, 0, 1))


# speedup vs baseline: 1.3536x; 1.0049x over previous
import functools

import jax
import jax.numpy as jnp
from jax import lax
from jax.experimental import pallas as pl
from jax.experimental.pallas import tpu as pltpu
from jax.experimental.pallas import tpu_sc as plsc

_DIM = 32
_NC = 2
_NS = 16
_NW = _NC * _NS
_K = 128
_NBUF = 8
_DELTA = 4


@functools.lru_cache(maxsize=None)
def _build(n, m, vocab):
    batch = n * m
    assert batch % (_NW * _K) == 0
    bpw = batch // _NW
    nch = bpw // _K
    cpc = n // _K
    assert (nch - _NBUF) % _NBUF == 0
    mesh = plsc.VectorSubcoreMesh(core_axis_name="c", subcore_axis_name="s")

    @functools.partial(
        pl.kernel,
        mesh=mesh,
        out_type=jax.ShapeDtypeStruct((_NW, nch, _K, _DIM), jnp.float32),
        scratch_types=(
            [pltpu.VMEM((nch, _K), jnp.int32),
             pltpu.VMEM((_NBUF, _K, _DIM), jnp.float32)]
            + [pltpu.SemaphoreType.DMA] * (2 * _NBUF)
        ),
        compiler_params=pltpu.CompilerParams(use_tc_tiling_on_sc=False),
    )
    def emb(idx_hbm, table_hbm, out_hbm, idx_v, rows_v, *sems):
        gsem = sems[:_NBUF]
        osem = sems[_NBUF:]
        wid = lax.axis_index("s") * _NC + lax.axis_index("c")
        pltpu.sync_copy(idx_hbm.at[wid], idx_v)

        def start_gather(c, b):
            pltpu.async_copy(table_hbm.at[idx_v.at[c]], rows_v.at[b], gsem[b])

        def wait_gather(c, b):
            pltpu.make_async_copy(
                table_hbm.at[idx_v.at[c]], rows_v.at[b], gsem[b]).wait()

        def start_out(c, b):
            pltpu.async_copy(rows_v.at[b], out_hbm.at[wid, c], osem[b])

        def wait_out(c, b):
            pltpu.make_async_copy(
                rows_v.at[b], out_hbm.at[wid, c], osem[b]).wait()

        for g in range(_DELTA):
            start_gather(g, g % _NBUF)
        for g in range(_DELTA, _NBUF):
            start_gather(g, g % _NBUF)
            c = g - _DELTA
            wait_gather(c, c % _NBUF)
            start_out(c, c % _NBUF)

        def group(i, carry):
            g0 = _NBUF + i * _NBUF
            for b in range(_NBUF):
                g = g0 + b
                wait_out(g - _NBUF, b)
                start_gather(g, b)
                cb = (b + _NBUF - _DELTA) % _NBUF
                wait_gather(g - _DELTA, cb)
                start_out(g - _DELTA, cb)
            return carry

        lax.fori_loop(0, (nch - _NBUF) // _NBUF, group, 0)

        for c in range(nch - _DELTA, nch):
            wait_gather(c, c % _NBUF)
            start_out(c, c % _NBUF)
        for c in range(nch - _NBUF, nch):
            wait_out(c, c % _NBUF)

    return emb


_TB = 8192


def _table_relayout(table_bc):
    vocab = table_bc.shape[1]
    grid = -(-vocab // _TB)

    def body(in_ref, out_ref):
        t = in_ref[...].T
        tp = jnp.pad(t, ((0, 0), (0, 128 - _DIM)))
        s = _TB // 4
        acc = tp[0:s]
        for q in range(1, 4):
            acc = acc + pltpu.roll(tp[q * s:(q + 1) * s], _DIM * q, axis=1)
        out_ref[...] = acc

    return pl.pallas_call(
        body,
        grid=(grid,),
        in_specs=[pl.BlockSpec((_DIM, _TB), lambda j: (0, j))],
        out_specs=pl.BlockSpec((_TB // 4, 128), lambda j: (j, 0)),
        out_shape=jax.ShapeDtypeStruct((grid * _TB // 4, 128),
                                       jnp.float32),
    )(table_bc)


_OB = 4


def _out_relayout(o):
    m, r128, _ = o.shape
    n = r128 * 128 // _DIM

    def body(in_ref, out_ref):
        for k in range(_OB):
            x = in_ref[k]
            pieces = []
            for q in range(4):
                r = pltpu.roll(x, 128 - _DIM * q, axis=1) if q else x
                pieces.append(r[:, None, :_DIM])
            emb = jnp.concatenate(pieces, axis=1).reshape(n, _DIM)
            out_ref[k] = emb.T

    return pl.pallas_call(
        body,
        grid=(m // _OB,),
        in_specs=[pl.BlockSpec((_OB, r128, 128), lambda j: (j, 0, 0))],
        out_specs=pl.BlockSpec((_OB, _DIM, n), lambda j: (j, 0, 0)),
        out_shape=jax.ShapeDtypeStruct((m, _DIM, n), jnp.float32),
    )(o)


def _remap_idx(idx):
    shape = idx.shape

    def body(in_ref, out_ref):
        v = in_ref[...]
        out_ref[...] = (v & -8192) + ((v & 2047) << 2) + ((v >> 11) & 3)

    return pl.pallas_call(
        body,
        in_specs=[pl.BlockSpec(shape, lambda: (0,) * len(shape))],
        out_specs=pl.BlockSpec(shape, lambda: (0,) * len(shape)),
        out_shape=jax.ShapeDtypeStruct(shape, jnp.int32),
        grid=(),
    )(idx)


def kernel(x, table):
    n, m = x.shape
    batch = n * m
    idx = _remap_idx(jnp.swapaxes(x, 0, 1).reshape(
        _NW, batch // _NW // _K, _K).astype(jnp.int32))
    t128 = _table_relayout(jnp.swapaxes(table, 0, 1))
    table_rm = t128.reshape(t128.shape[0] * 128 // _DIM, _DIM)
    out = _build(n, m, table_rm.shape[0])(idx, table_rm)
    ot = _out_relayout(out.reshape(m, n * _DIM // 128, 128))
    return jnp.transpose(ot, (2, 0, 1))
